# baseline (speedup 1.0000x reference)
import jax
import jax.numpy as jnp
from jax import lax
from jax.experimental import pallas as pl
from jax.experimental.pallas import tpu as pltpu
from jax.experimental.pallas import tpu_sc as plsc

N = 100000
E = 1600000
G = 1000

NC = 2
NS = 16
LANES = 16

K_ACC = 100352
ZROWS = 64
KCH = 5
ROWS_E = 12800
ROWS_P = 800
K_POOL = 1024

_mesh = plsc.VectorSubcoreMesh(core_axis_name="c", subcore_axis_name="s")


def _sc_pass(mode, rows, spc, k_acc, out_s, spc_base=0):
    gather = mode != "deg"
    stride = 8
    if mode == "deg":
        rows_per_tile = rows // (NC * NS)
    else:
        rows_per_tile = rows // NS
    n_chunks = rows_per_tile // KCH
    assert rows_per_tile % KCH == 0
    stripe = k_acc // NS
    n_zcopy = stripe // ZROWS
    assert stripe % ZROWS == 0

    scratch = [
        pltpu.VMEM((3, KCH, 128), jnp.int32),
        pltpu.VMEM((min(ZROWS, stripe), LANES), jnp.float32),
        pltpu.SemaphoreType.DMA,
        pltpu.SemaphoreType.DMA,
    ]
    if gather:
        scratch += [
            pltpu.VMEM((2, KCH, 128), jnp.int32),
            pltpu.VMEM((2, KCH, 128, LANES), jnp.float32),
            pltpu.SemaphoreType.DMA,
        ]
    else:
        scratch += [pltpu.VMEM((128, LANES), jnp.float32)]
    scratch.append(pltpu.VMEM_SHARED((k_acc, LANES), jnp.float32))

    del out_s
    out_type = jax.ShapeDtypeStruct((k_acc, 8, LANES), jnp.float32)


    def body(*refs):
        if mode == "edge":
            table, srcr, dstr, out = refs[:4]
            dbuf, zbuf, isem, ssem, sbuf, rbuf, gsem, acc = refs[4:]
        elif mode == "pool":
            table, dstr, out = refs[:3]
            dbuf, zbuf, isem, ssem, sbuf, rbuf, gsem, acc = refs[3:]
        else:
            dstr, out = refs[:2]
            dbuf, zbuf, isem, ssem, obuf, acc = refs[2:]
        cid = lax.axis_index("c")
        sid = lax.axis_index("s")
        iota16 = lax.iota(jnp.int32, 16)

        zn = min(ZROWS, stripe)
        @pl.loop(0, zn)
        def _(i):
            zbuf[i, :] = jnp.zeros((LANES,), jnp.float32)
        if not gather:
            @pl.loop(0, 128)
            def _(i):
                obuf[i, :] = jnp.ones((LANES,), jnp.float32)

        if mode == "deg":
            row0 = (cid * NS + sid) * rows_per_tile
        else:
            row0 = sid * rows_per_tile

        def fire_idx(t):
            p3 = lax.rem(t, 3)
            rbase = row0 + t * KCH
            h = [pltpu.async_copy(dstr.at[pl.ds(rbase, KCH)], dbuf.at[p3], isem)]
            if mode == "edge":
                p2 = lax.rem(t, 2)
                h.append(pltpu.async_copy(srcr.at[pl.ds(rbase, KCH)],
                                          sbuf.at[p2], isem))
            return h

        def wait_idx(t):
            p3 = lax.rem(t, 3)
            rbase = row0 + t * KCH
            pltpu.make_async_copy(dstr.at[pl.ds(rbase, KCH)], dbuf.at[p3],
                                  isem).wait()
            if mode == "edge":
                p2 = lax.rem(t, 2)
                pltpu.make_async_copy(srcr.at[pl.ds(rbase, KCH)], sbuf.at[p2],
                                      isem).wait()

        def stage_b(t, s):
            wait_idx(t)
            if not gather:
                return
            p2 = lax.rem(t, 2)
            if mode == "edge":
                for j in range(KCH):
                    for g in range(8):
                        v = sbuf[p2, j, pl.ds(g * 16, 16)]
                        sbuf[p2, j, pl.ds(g * 16, 16)] = v * stride + s
            else:
                rbase = row0 + t * KCH
                for j in range(KCH):
                    for g in range(8):
                        vid = (rbase + j) * 128 + g * 16 + iota16
                        vid = jnp.minimum(vid, N - 1)
                        sbuf[p2, j, pl.ds(g * 16, 16)] = vid * stride + s
            for j in range(KCH):
                pltpu.async_copy(table.at[sbuf.at[p2, j]], rbuf.at[p2, j], gsem)

        def stage_c(t):
            p2 = lax.rem(t, 2)
            p3 = lax.rem(t, 3)
            for j in range(KCH):
                if gather:
                    pltpu.make_async_copy(table.at[sbuf.at[p2, j]],
                                          rbuf.at[p2, j], gsem).wait()
                    src_rows = rbuf.at[p2, j]
                else:
                    src_rows = obuf
                pltpu.async_copy(src_rows, acc.at[dbuf.at[p3, j]], ssem,
                                 add=True)

        def stage_d(t):
            p2 = lax.rem(t, 2)
            p3 = lax.rem(t, 3)
            for j in range(KCH):
                src_rows = rbuf.at[p2, j] if gather else obuf
                pltpu.make_async_copy(src_rows, acc.at[dbuf.at[p3, j]],
                                      ssem).wait()

        for sl in range(spc):
            s = spc_base + (cid * spc + sl) if mode == "edge" else cid

            @pl.loop(0, n_zcopy)
            def _(i):
                pltpu.sync_copy(zbuf, acc.at[pl.ds(sid * stripe + i * zn, zn)])
            plsc.subcore_barrier()

            fire_idx(0)

            @pl.loop(0, n_chunks + 2)
            def _(c):
                @pl.when(c >= 2)
                def _():
                    stage_d(c - 2)
                if gather:
                    @pl.when((c >= 1) & (c <= n_chunks))
                    def _():
                        stage_c(c - 1)
                @pl.when(c + 1 <= n_chunks - 1)
                def _():
                    fire_idx(c + 1)
                @pl.when(c <= n_chunks - 1)
                def _():
                    if gather:
                        stage_b(c, s)
                    else:
                        wait_idx(c)
                        stage_c(c)
            plsc.subcore_barrier()

            @pl.loop(0, n_zcopy)
            def _(i):
                off = sid * stripe + i * zn
                pltpu.sync_copy(acc.at[pl.ds(off, zn)],
                                out.at[pl.ds(off, zn), s])
            plsc.subcore_barrier()

    return pl.kernel(
        body, out_type=out_type, mesh=_mesh, scratch_types=scratch,
        compiler_params=pltpu.CompilerParams(use_tc_tiling_on_sc=False),
    )


NP = K_ACC
RB = 1024
NB = NP // RB
TB = RB // 8
ECOLS = 16384


def _idx_kernel(e_ref, src_ref, dst_ref):
    i = pl.program_id(0)
    f = (i * ECOLS
         + lax.broadcasted_iota(jnp.int32, (128, 128), 0) * 128
         + lax.broadcasted_iota(jnp.int32, (128, 128), 1))
    mask = f < E
    src_ref[...] = jnp.where(mask, e_ref[0].reshape(128, 128),
                             lax.rem(f, N))
    dst_ref[...] = jnp.where(mask, e_ref[1].reshape(128, 128),
                             N + (f & 15))


def _deg_dinv(dp_ref):
    m = dp_ref[...][:, :, :32].reshape(RB, 32)
    return lax.rsqrt(m[:, 0:1] + m[:, 16:17] + 1.0)


def _pad128(v):
    r, c = v.shape
    return jnp.concatenate(
        [v.reshape(r // 8, 8, c),
         jnp.zeros((r // 8, 8, 128 - c), jnp.float32)], axis=2)


def _mm1_kernel(x_ref, w_ref, o_ref):
    o_ref[...] = jnp.dot(x_ref[...], w_ref[...],
                         preferred_element_type=jnp.float32)


def _prep_kernel(dp_ref, xw_ref, y_ref):
    dinv = _deg_dinv(dp_ref)
    y_ref[...] = _pad128(xw_ref[...] * dinv)


def _layer1a_kernel(agg_ref, y_ref, dp_ref, b_ref, w_ref, pa_ref):
    agg = agg_ref[...][:, :, :32].reshape(RB, 32)
    y = y_ref[...][:, :, :32].reshape(RB, 32)
    dinv = _deg_dinv(dp_ref)
    h = jnp.maximum(dinv * (agg + y) + b_ref[...], 0.0)
    pa_ref[...] = jnp.dot(h, w_ref[...], preferred_element_type=jnp.float32)


def _layer1b_kernel(agg_ref, y_ref, dp_ref, pa_ref, b_ref, w_ref, y2_ref):
    agg = agg_ref[...][:, :, 32:64].reshape(RB, 32)
    y = y_ref[...][:, :, 32:64].reshape(RB, 32)
    dinv = _deg_dinv(dp_ref)
    h = jnp.maximum(dinv * (agg + y) + b_ref[...], 0.0)
    y2 = (pa_ref[...]
          + jnp.dot(h, w_ref[...], preferred_element_type=jnp.float32)) * dinv
    y2_ref[...] = _pad128(y2)


def _h2_kernel(agg_ref, y_ref, dp_ref, b_ref, h_ref):
    agg = agg_ref[...][:, :, :32].reshape(RB, 32)
    y = y_ref[...][:, :, :32].reshape(RB, 32)
    dinv = _deg_dinv(dp_ref)
    h_ref[...] = _pad128(jnp.maximum(dinv * (agg + y) + b_ref[...], 0.0))


def _mlp_kernel(gp_ref, w1, b1, w2, b2, w3, b3, w4, b4, o_ref):
    g = gp_ref[...][:, :, :32].reshape(K_POOL, 32)[:G]
    g = jnp.maximum(jnp.dot(g, w1[...], preferred_element_type=jnp.float32)
                    + b1[...], 0.0)
    g = jnp.maximum(jnp.dot(g, w2[...], preferred_element_type=jnp.float32)
                    + b2[...], 0.0)
    g = jnp.maximum(jnp.dot(g, w3[...], preferred_element_type=jnp.float32)
                    + b3[...], 0.0)
    o_ref[...] = jnp.dot(g, w4[...], preferred_element_type=jnp.float32) + b4[...]


def _full(shape):
    return pl.BlockSpec(shape, lambda i: tuple(0 for _ in shape))


def _rows(shape):
    return pl.BlockSpec(shape, lambda i: (i,) + tuple(0 for _ in shape[1:]))


def kernel(x, edge_index, batch, W1, b1, W2, b2, RW1, Rb1, RW2, Rb2, RW3, Rb3, RW4, Rb4):
    f32 = jnp.float32
    ei32 = edge_index.astype(jnp.int32)
    batch32 = batch.astype(jnp.int32)

    src_rows, dst_rows = pl.pallas_call(
        _idx_kernel, grid=(ROWS_E // 128,),
        in_specs=[pl.BlockSpec((2, 128, 128),
                               lambda i: (0, jnp.minimum(i, E // ECOLS), 0))],
        out_specs=[_rows((128, 128)), _rows((128, 128))],
        out_shape=[jax.ShapeDtypeStruct((ROWS_E, 128), jnp.int32),
                   jax.ShapeDtypeStruct((ROWS_E, 128), jnp.int32)],
    )(ei32.reshape(2, E // 128, 128))

    n_pad_p = ROWS_P * 128 - N
    iot_p = lax.iota(jnp.int32, n_pad_p)
    pdst_rows = jnp.concatenate(
        [batch32, (G + 8) + (iot_p % 16)]).reshape(ROWS_P, 128)

    deg_pass = _sc_pass("deg", ROWS_E, 1, K_ACC, 2)
    agg4a_pass = _sc_pass("edge", ROWS_E, 1, K_ACC, 2, spc_base=0)
    agg4b_pass = _sc_pass("edge", ROWS_E, 1, K_ACC, 2, spc_base=2)
    agg2_pass = _sc_pass("edge", ROWS_E, 1, K_ACC, 2)
    pool_pass = _sc_pass("pool", ROWS_P, 1, K_POOL, 2)

    deg_parts = deg_pass(dst_rows)
    deg_r = deg_parts.reshape(NP // 8, 8, 128)
    xw1 = pl.pallas_call(
        _mm1_kernel, grid=(NB,),
        in_specs=[_rows((RB, 47)), _full((47, 64))],
        out_specs=_rows((RB, 64)),
        out_shape=jax.ShapeDtypeStruct((NP, 64), f32),
    )(x, W1)

    y1p = pl.pallas_call(
        _prep_kernel, grid=(NB,),
        in_specs=[_rows((TB, 8, 128)), _rows((RB, 64))],
        out_specs=_rows((TB, 8, 128)),
        out_shape=jax.ShapeDtypeStruct((NP // 8, 8, 128), f32),
    )(deg_r, xw1)

    t1 = y1p.reshape(NP * 8, 16)
    agg1a = agg4a_pass(t1, src_rows, dst_rows)
    agg1b = agg4b_pass(t1, src_rows, dst_rows)
    partA = pl.pallas_call(
        _layer1a_kernel, grid=(NB,),
        in_specs=[_rows((TB, 8, 128)), _rows((TB, 8, 128)),
                  _rows((TB, 8, 128)), _full((1, 32)), _full((32, 32))],
        out_specs=_rows((RB, 32)),
        out_shape=jax.ShapeDtypeStruct((NP, 32), f32),
    )(agg1a.reshape(NP // 8, 8, 128), y1p, deg_r,
      b1[:32].reshape(1, 32), W2[:32])
    y2p = pl.pallas_call(
        _layer1b_kernel, grid=(NB,),
        in_specs=[_rows((TB, 8, 128)), _rows((TB, 8, 128)),
                  _rows((TB, 8, 128)), _rows((RB, 32)),
                  _full((1, 32)), _full((32, 32))],
        out_specs=_rows((TB, 8, 128)),
        out_shape=jax.ShapeDtypeStruct((NP // 8, 8, 128), f32),
    )(agg1b.reshape(NP // 8, 8, 128), y1p, deg_r, partA,
      b1[32:].reshape(1, 32), W2[32:])

    t2 = y2p.reshape(NP * 8, 16)
    agg2 = agg2_pass(t2, src_rows, dst_rows)
    h2p = pl.pallas_call(
        _h2_kernel, grid=(NB,),
        in_specs=[_rows((TB, 8, 128)), _rows((TB, 8, 128)),
                  _rows((TB, 8, 128)), _full((1, 32))],
        out_specs=_rows((TB, 8, 128)),
        out_shape=jax.ShapeDtypeStruct((NP // 8, 8, 128), f32),
    )(agg2.reshape(NP // 8, 8, 128), y2p, deg_r, b2.reshape(1, 32))

    tp = h2p.reshape(NP * 8, 16)
    gp = pool_pass(tp, pdst_rows)
    out = pl.pallas_call(
        _mlp_kernel, grid=(1,),
        in_specs=[_full((K_POOL // 8, 8, 128)),
                  _full((32, 32)), _full((1, 32)),
                  _full((32, 16)), _full((1, 16)),
                  _full((16, 8)), _full((1, 8)),
                  _full((8, 1)), _full((1, 1))],
        out_specs=_full((G, 1)),
        out_shape=jax.ShapeDtypeStruct((G, 1), f32),
    )(gp.reshape(K_POOL // 8, 8, 128), RW1, Rb1.reshape(1, 32), RW2,
      Rb2.reshape(1, 16), RW3, Rb3.reshape(1, 8), RW4, Rb4.reshape(1, 1))
    return out


# speedup vs baseline: 1.0267x; 1.0267x over previous
import jax
import jax.numpy as jnp
from jax import lax
from jax.experimental import pallas as pl
from jax.experimental.pallas import tpu as pltpu
from jax.experimental.pallas import tpu_sc as plsc

N = 100000
E = 1600000
G = 1000

NC = 2
NS = 16
LANES = 16

K_ACC = 100352
ZROWS = 64
KCH = 5
ROWS_E = 12800
ROWS_P = 800
K_POOL = 1024

_mesh = plsc.VectorSubcoreMesh(core_axis_name="c", subcore_axis_name="s")


def _sc_pass(mode, rows, spc, k_acc, out_s, spc_base=0):
    gather = mode != "deg"
    stride = 8
    if mode == "deg":
        rows_per_tile = rows // (NC * NS)
    else:
        rows_per_tile = rows // NS
    n_chunks = rows_per_tile // KCH
    assert rows_per_tile % KCH == 0
    stripe = k_acc // NS
    n_zcopy = stripe // ZROWS
    assert stripe % ZROWS == 0

    scratch = [
        pltpu.VMEM((3, KCH, 128), jnp.int32),
        pltpu.VMEM((min(ZROWS, stripe), LANES), jnp.float32),
        pltpu.SemaphoreType.DMA,
        pltpu.SemaphoreType.DMA,
    ]
    if gather:
        scratch += [
            pltpu.VMEM((2, KCH, 128), jnp.int32),
            pltpu.VMEM((2, KCH, 128, LANES), jnp.float32),
            pltpu.SemaphoreType.DMA,
        ]
    else:
        scratch += [pltpu.VMEM((128, LANES), jnp.float32)]
    scratch.append(pltpu.VMEM_SHARED((k_acc, LANES), jnp.float32))

    del out_s
    out_type = jax.ShapeDtypeStruct((k_acc, 8, LANES), jnp.float32)


    def body(*refs):
        if mode == "edge":
            table, srcr, dstr, out = refs[:4]
            dbuf, zbuf, isem, ssem, sbuf, rbuf, gsem, acc = refs[4:]
        elif mode == "pool":
            table, dstr, out = refs[:3]
            dbuf, zbuf, isem, ssem, sbuf, rbuf, gsem, acc = refs[3:]
        else:
            dstr, out = refs[:2]
            dbuf, zbuf, isem, ssem, obuf, acc = refs[2:]
        cid = lax.axis_index("c")
        sid = lax.axis_index("s")
        iota16 = lax.iota(jnp.int32, 16)

        zn = min(ZROWS, stripe)
        @pl.loop(0, zn)
        def _(i):
            zbuf[i, :] = jnp.zeros((LANES,), jnp.float32)
        if not gather:
            @pl.loop(0, 128)
            def _(i):
                obuf[i, :] = jnp.ones((LANES,), jnp.float32)

        if mode == "deg":
            row0 = (cid * NS + sid) * rows_per_tile
        else:
            row0 = sid * rows_per_tile

        def fire_idx(t):
            p3 = lax.rem(t, 3)
            rbase = row0 + t * KCH
            h = [pltpu.async_copy(dstr.at[pl.ds(rbase, KCH)], dbuf.at[p3], isem)]
            if mode == "edge":
                p2 = lax.rem(t, 2)
                h.append(pltpu.async_copy(srcr.at[pl.ds(rbase, KCH)],
                                          sbuf.at[p2], isem))
            return h

        def wait_idx(t):
            p3 = lax.rem(t, 3)
            rbase = row0 + t * KCH
            pltpu.make_async_copy(dstr.at[pl.ds(rbase, KCH)], dbuf.at[p3],
                                  isem).wait()
            if mode == "edge":
                p2 = lax.rem(t, 2)
                pltpu.make_async_copy(srcr.at[pl.ds(rbase, KCH)], sbuf.at[p2],
                                      isem).wait()

        def stage_b(t, s):
            wait_idx(t)
            if not gather:
                return
            p2 = lax.rem(t, 2)
            if mode == "edge":
                for j in range(KCH):
                    for g in range(8):
                        v = sbuf[p2, j, pl.ds(g * 16, 16)]
                        sbuf[p2, j, pl.ds(g * 16, 16)] = v * stride + s
            else:
                rbase = row0 + t * KCH
                for j in range(KCH):
                    for g in range(8):
                        vid = (rbase + j) * 128 + g * 16 + iota16
                        vid = jnp.minimum(vid, N - 1)
                        sbuf[p2, j, pl.ds(g * 16, 16)] = vid * stride + s
            for j in range(KCH):
                pltpu.async_copy(table.at[sbuf.at[p2, j]], rbuf.at[p2, j], gsem)

        def stage_c(t):
            p2 = lax.rem(t, 2)
            p3 = lax.rem(t, 3)
            for j in range(KCH):
                if gather:
                    pltpu.make_async_copy(table.at[sbuf.at[p2, j]],
                                          rbuf.at[p2, j], gsem).wait()
                    src_rows = rbuf.at[p2, j]
                else:
                    src_rows = obuf
                pltpu.async_copy(src_rows, acc.at[dbuf.at[p3, j]], ssem,
                                 add=True)

        def stage_d(t):
            p2 = lax.rem(t, 2)
            p3 = lax.rem(t, 3)
            for j in range(KCH):
                src_rows = rbuf.at[p2, j] if gather else obuf
                pltpu.make_async_copy(src_rows, acc.at[dbuf.at[p3, j]],
                                      ssem).wait()

        for sl in range(spc):
            s = spc_base + (cid * spc + sl) if mode == "edge" else cid

            @pl.loop(0, n_zcopy)
            def _(i):
                pltpu.sync_copy(zbuf, acc.at[pl.ds(sid * stripe + i * zn, zn)])
            plsc.subcore_barrier()

            fire_idx(0)

            @pl.loop(0, n_chunks + 2)
            def _(c):
                @pl.when(c >= 2)
                def _():
                    stage_d(c - 2)
                if gather:
                    @pl.when((c >= 1) & (c <= n_chunks))
                    def _():
                        stage_c(c - 1)
                @pl.when(c + 1 <= n_chunks - 1)
                def _():
                    fire_idx(c + 1)
                @pl.when(c <= n_chunks - 1)
                def _():
                    if gather:
                        stage_b(c, s)
                    else:
                        wait_idx(c)
                        stage_c(c)
            plsc.subcore_barrier()

            @pl.loop(0, n_zcopy)
            def _(i):
                off = sid * stripe + i * zn
                pltpu.sync_copy(acc.at[pl.ds(off, zn)],
                                out.at[pl.ds(off, zn), s])
            plsc.subcore_barrier()

    return pl.kernel(
        body, out_type=out_type, mesh=_mesh, scratch_types=scratch,
        compiler_params=pltpu.CompilerParams(use_tc_tiling_on_sc=False),
    )


NP = K_ACC
RB = 1024
NB = NP // RB
TB = RB // 8
ECOLS = 16384


def _idx_kernel(e_ref, src_ref, dst_ref):
    i = pl.program_id(0)
    f = (i * ECOLS
         + lax.broadcasted_iota(jnp.int32, (128, 128), 0) * 128
         + lax.broadcasted_iota(jnp.int32, (128, 128), 1))
    mask = f < E
    src_ref[...] = jnp.where(mask, e_ref[0].reshape(128, 128),
                             lax.rem(f, N))
    dst_ref[...] = jnp.where(mask, e_ref[1].reshape(128, 128),
                             N + (f & 15))


def _deg_dinv(dp_ref):
    m = dp_ref[...][:, :, :32].reshape(RB, 32)
    return lax.rsqrt(m[:, 0:1] + m[:, 16:17] + 1.0)


def _pad128(v):
    r, c = v.shape
    return jnp.concatenate(
        [v.reshape(r // 8, 8, c),
         jnp.zeros((r // 8, 8, 128 - c), jnp.float32)], axis=2)


def _mm1_kernel(x_ref, w_ref, o_ref):
    o_ref[...] = jnp.dot(x_ref[...], w_ref[...],
                         preferred_element_type=jnp.float32)


def _prep_kernel(dp_ref, xw_ref, y_ref):
    dinv = _deg_dinv(dp_ref)
    y_ref[...] = _pad128(xw_ref[...] * dinv)


def _layer1_kernel(agg_ref, y_ref, dp_ref, b_ref, w_ref, y2_ref):
    agg = agg_ref[...][:, :, :64].reshape(RB, 64)
    y = y_ref[...][:, :, :64].reshape(RB, 64)
    dinv = _deg_dinv(dp_ref)
    h = jnp.maximum(dinv * (agg + y) + b_ref[...], 0.0)
    y2 = jnp.dot(h, w_ref[...], preferred_element_type=jnp.float32) * dinv
    y2_ref[...] = _pad128(y2)


def _h2_kernel(agg_ref, y_ref, dp_ref, b_ref, h_ref):
    agg = agg_ref[...][:, :, :32].reshape(RB, 32)
    y = y_ref[...][:, :, :32].reshape(RB, 32)
    dinv = _deg_dinv(dp_ref)
    h_ref[...] = _pad128(jnp.maximum(dinv * (agg + y) + b_ref[...], 0.0))


def _mlp_kernel(gp_ref, w1, b1, w2, b2, w3, b3, w4, b4, o_ref):
    g = gp_ref[...][:, :, :32].reshape(K_POOL, 32)[:G]
    g = jnp.maximum(jnp.dot(g, w1[...], preferred_element_type=jnp.float32)
                    + b1[...], 0.0)
    g = jnp.maximum(jnp.dot(g, w2[...], preferred_element_type=jnp.float32)
                    + b2[...], 0.0)
    g = jnp.maximum(jnp.dot(g, w3[...], preferred_element_type=jnp.float32)
                    + b3[...], 0.0)
    o_ref[...] = jnp.dot(g, w4[...], preferred_element_type=jnp.float32) + b4[...]


def _full(shape):
    return pl.BlockSpec(shape, lambda i: tuple(0 for _ in shape))


def _rows(shape):
    return pl.BlockSpec(shape, lambda i: (i,) + tuple(0 for _ in shape[1:]))


def kernel(x, edge_index, batch, W1, b1, W2, b2, RW1, Rb1, RW2, Rb2, RW3, Rb3, RW4, Rb4):
    f32 = jnp.float32
    ei32 = edge_index.astype(jnp.int32)
    batch32 = batch.astype(jnp.int32)

    src_rows, dst_rows = pl.pallas_call(
        _idx_kernel, grid=(ROWS_E // 128,),
        in_specs=[pl.BlockSpec((2, 128, 128),
                               lambda i: (0, jnp.minimum(i, E // ECOLS), 0))],
        out_specs=[_rows((128, 128)), _rows((128, 128))],
        out_shape=[jax.ShapeDtypeStruct((ROWS_E, 128), jnp.int32),
                   jax.ShapeDtypeStruct((ROWS_E, 128), jnp.int32)],
    )(ei32.reshape(2, E // 128, 128))

    n_pad_p = ROWS_P * 128 - N
    iot_p = lax.iota(jnp.int32, n_pad_p)
    pdst_rows = jnp.concatenate(
        [batch32, (G + 8) + (iot_p % 16)]).reshape(ROWS_P, 128)

    deg_pass = _sc_pass("deg", ROWS_E, 1, K_ACC, 2)
    agg4_pass = _sc_pass("edge", ROWS_E, 2, K_ACC, 4)
    agg2_pass = _sc_pass("edge", ROWS_E, 1, K_ACC, 2)
    pool_pass = _sc_pass("pool", ROWS_P, 1, K_POOL, 2)

    deg_parts = deg_pass(dst_rows)
    deg_r = deg_parts.reshape(NP // 8, 8, 128)
    xw1 = pl.pallas_call(
        _mm1_kernel, grid=(NB,),
        in_specs=[_rows((RB, 47)), _full((47, 64))],
        out_specs=_rows((RB, 64)),
        out_shape=jax.ShapeDtypeStruct((NP, 64), f32),
    )(x, W1)

    y1p = pl.pallas_call(
        _prep_kernel, grid=(NB,),
        in_specs=[_rows((TB, 8, 128)), _rows((RB, 64))],
        out_specs=_rows((TB, 8, 128)),
        out_shape=jax.ShapeDtypeStruct((NP // 8, 8, 128), f32),
    )(deg_r, xw1)

    t1 = y1p.reshape(NP * 8, 16)
    agg1 = agg4_pass(t1, src_rows, dst_rows)
    y2p = pl.pallas_call(
        _layer1_kernel, grid=(NB,),
        in_specs=[_rows((TB, 8, 128)), _rows((TB, 8, 128)),
                  _rows((TB, 8, 128)), _full((1, 64)), _full((64, 32))],
        out_specs=_rows((TB, 8, 128)),
        out_shape=jax.ShapeDtypeStruct((NP // 8, 8, 128), f32),
    )(agg1.reshape(NP // 8, 8, 128), y1p, deg_r, b1.reshape(1, 64), W2)

    t2 = y2p.reshape(NP * 8, 16)
    agg2 = agg2_pass(t2, src_rows, dst_rows)
    h2p = pl.pallas_call(
        _h2_kernel, grid=(NB,),
        in_specs=[_rows((TB, 8, 128)), _rows((TB, 8, 128)),
                  _rows((TB, 8, 128)), _full((1, 32))],
        out_specs=_rows((TB, 8, 128)),
        out_shape=jax.ShapeDtypeStruct((NP // 8, 8, 128), f32),
    )(agg2.reshape(NP // 8, 8, 128), y2p, deg_r, b2.reshape(1, 32))

    tp = h2p.reshape(NP * 8, 16)
    gp = pool_pass(tp, pdst_rows)
    out = pl.pallas_call(
        _mlp_kernel, grid=(1,),
        in_specs=[_full((K_POOL // 8, 8, 128)),
                  _full((32, 32)), _full((1, 32)),
                  _full((32, 16)), _full((1, 16)),
                  _full((16, 8)), _full((1, 8)),
                  _full((8, 1)), _full((1, 1))],
        out_specs=_full((G, 1)),
        out_shape=jax.ShapeDtypeStruct((G, 1), f32),
    )(gp.reshape(K_POOL // 8, 8, 128), RW1, Rb1.reshape(1, 32), RW2,
      Rb2.reshape(1, 16), RW3, Rb3.reshape(1, 8), RW4, Rb4.reshape(1, 1))
    return out


# speedup vs baseline: 1.0897x; 1.0614x over previous
import jax
import jax.numpy as jnp
from jax import lax
from jax.experimental import pallas as pl
from jax.experimental.pallas import tpu as pltpu
from jax.experimental.pallas import tpu_sc as plsc

N = 100000
E = 1600000
G = 1000

NC = 2
NS = 16
LANES = 16

K_ACC = 100352
ZROWS = 64
KCH = 5
ROWS_E = 12800
ROWS_P = 800
K_POOL = 1024

_mesh = plsc.VectorSubcoreMesh(core_axis_name="c", subcore_axis_name="s")


def _sc_pass(mode, rows, spc, k_acc, out_s, spc_base=0):
    gather = mode != "deg"
    stride = 8
    if mode == "deg":
        rows_per_tile = rows // (NC * NS)
    else:
        rows_per_tile = rows // NS
    n_chunks = rows_per_tile // KCH
    assert rows_per_tile % KCH == 0
    stripe = k_acc // NS
    n_zcopy = stripe // ZROWS
    assert stripe % ZROWS == 0

    scratch = [
        pltpu.VMEM((3, KCH, 128), jnp.int32),
        pltpu.VMEM((min(ZROWS, stripe), LANES), jnp.float32),
        pltpu.SemaphoreType.DMA,
        pltpu.SemaphoreType.DMA,
    ]
    if gather:
        scratch += [
            pltpu.VMEM((2, KCH, 128), jnp.int32),
            pltpu.VMEM((2, KCH, 128, LANES), jnp.float32),
            pltpu.SemaphoreType.DMA,
        ]
    else:
        scratch += [pltpu.VMEM((128, LANES), jnp.float32)]
    scratch.append(pltpu.VMEM_SHARED((k_acc, LANES), jnp.float32))

    del out_s
    out_type = jax.ShapeDtypeStruct((k_acc, 8, LANES), jnp.float32)


    def body(*refs):
        if mode == "edge":
            table, srcr, dstr, out = refs[:4]
            dbuf, zbuf, isem, ssem, sbuf, rbuf, gsem, acc = refs[4:]
        elif mode == "pool":
            table, dstr, out = refs[:3]
            dbuf, zbuf, isem, ssem, sbuf, rbuf, gsem, acc = refs[3:]
        else:
            dstr, out = refs[:2]
            dbuf, zbuf, isem, ssem, obuf, acc = refs[2:]
        cid = lax.axis_index("c")
        sid = lax.axis_index("s")
        iota16 = lax.iota(jnp.int32, 16)

        zn = min(ZROWS, stripe)
        @pl.loop(0, zn)
        def _(i):
            zbuf[i, :] = jnp.zeros((LANES,), jnp.float32)
        if not gather:
            @pl.loop(0, 128)
            def _(i):
                obuf[i, :] = jnp.ones((LANES,), jnp.float32)

        if mode == "deg":
            row0 = (cid * NS + sid) * rows_per_tile
        else:
            row0 = sid * rows_per_tile

        def fire_idx(t):
            p3 = lax.rem(t, 3)
            rbase = row0 + t * KCH
            h = [pltpu.async_copy(dstr.at[pl.ds(rbase, KCH)], dbuf.at[p3], isem)]
            if mode == "edge":
                p2 = lax.rem(t, 2)
                h.append(pltpu.async_copy(srcr.at[pl.ds(rbase, KCH)],
                                          sbuf.at[p2], isem))
            return h

        def wait_idx(t):
            p3 = lax.rem(t, 3)
            rbase = row0 + t * KCH
            pltpu.make_async_copy(dstr.at[pl.ds(rbase, KCH)], dbuf.at[p3],
                                  isem).wait()
            if mode == "edge":
                p2 = lax.rem(t, 2)
                pltpu.make_async_copy(srcr.at[pl.ds(rbase, KCH)], sbuf.at[p2],
                                      isem).wait()

        def stage_b(t, s):
            wait_idx(t)
            if not gather:
                return
            p2 = lax.rem(t, 2)
            if mode == "edge":
                for j in range(KCH):
                    for g in range(8):
                        v = sbuf[p2, j, pl.ds(g * 16, 16)]
                        sbuf[p2, j, pl.ds(g * 16, 16)] = v * stride + s
            else:
                rbase = row0 + t * KCH
                for j in range(KCH):
                    for g in range(8):
                        vid = (rbase + j) * 128 + g * 16 + iota16
                        vid = jnp.minimum(vid, N - 1)
                        sbuf[p2, j, pl.ds(g * 16, 16)] = vid * stride + s
            for j in range(KCH):
                pltpu.async_copy(table.at[sbuf.at[p2, j]], rbuf.at[p2, j], gsem)

        def stage_c(t):
            p2 = lax.rem(t, 2)
            p3 = lax.rem(t, 3)
            for j in range(KCH):
                if gather:
                    pltpu.make_async_copy(table.at[sbuf.at[p2, j]],
                                          rbuf.at[p2, j], gsem).wait()
                    src_rows = rbuf.at[p2, j]
                else:
                    src_rows = obuf
                pltpu.async_copy(src_rows, acc.at[dbuf.at[p3, j]], ssem,
                                 add=True)

        def stage_d(t):
            p2 = lax.rem(t, 2)
            p3 = lax.rem(t, 3)
            for j in range(KCH):
                src_rows = rbuf.at[p2, j] if gather else obuf
                pltpu.make_async_copy(src_rows, acc.at[dbuf.at[p3, j]],
                                      ssem).wait()

        for sl in range(spc):
            s = spc_base + (cid * spc + sl) if mode == "edge" else cid

            @pl.loop(0, n_zcopy)
            def _(i):
                pltpu.sync_copy(zbuf, acc.at[pl.ds(sid * stripe + i * zn, zn)])
            plsc.subcore_barrier()

            fire_idx(0)

            @pl.loop(0, n_chunks + 2)
            def _(c):
                @pl.when(c >= 2)
                def _():
                    stage_d(c - 2)
                if gather:
                    @pl.when((c >= 1) & (c <= n_chunks))
                    def _():
                        stage_c(c - 1)
                @pl.when(c + 1 <= n_chunks - 1)
                def _():
                    fire_idx(c + 1)
                @pl.when(c <= n_chunks - 1)
                def _():
                    if gather:
                        stage_b(c, s)
                    else:
                        wait_idx(c)
                        stage_c(c)
            plsc.subcore_barrier()

            @pl.loop(0, n_zcopy)
            def _(i):
                off = sid * stripe + i * zn
                pltpu.sync_copy(acc.at[pl.ds(off, zn)],
                                out.at[pl.ds(off, zn), s])
            plsc.subcore_barrier()

    return pl.kernel(
        body, out_type=out_type, mesh=_mesh, scratch_types=scratch,
        compiler_params=pltpu.CompilerParams(use_tc_tiling_on_sc=False),
    )


NP = K_ACC
RB = 2048
NB = NP // RB
TB = RB // 8
ECOLS = 16384


def _idx_kernel(e_ref, src_ref, dst_ref):
    i = pl.program_id(0)
    f = (i * ECOLS
         + lax.broadcasted_iota(jnp.int32, (128, 128), 0) * 128
         + lax.broadcasted_iota(jnp.int32, (128, 128), 1))
    mask = f < E
    src_ref[...] = jnp.where(mask, e_ref[0].reshape(128, 128),
                             lax.rem(f, N))
    dst_ref[...] = jnp.where(mask, e_ref[1].reshape(128, 128),
                             N + (f & 15))


def _deg_dinv(dp_ref):
    m = dp_ref[...][:, :, :32].reshape(RB, 32)
    return lax.rsqrt(m[:, 0:1] + m[:, 16:17] + 1.0)


def _pad128(v):
    r, c = v.shape
    return jnp.concatenate(
        [v.reshape(r // 8, 8, c),
         jnp.zeros((r // 8, 8, 128 - c), jnp.float32)], axis=2)


def _mm1_kernel(x_ref, w_ref, o_ref):
    o_ref[...] = jnp.dot(x_ref[...], w_ref[...],
                         preferred_element_type=jnp.float32)


def _prep_kernel(dp_ref, xw_ref, y_ref):
    dinv = _deg_dinv(dp_ref)
    y_ref[...] = _pad128(xw_ref[...] * dinv)


def _layer1_kernel(agg_ref, y_ref, dp_ref, b_ref, w_ref, y2_ref):
    agg = agg_ref[...][:, :, :64].reshape(RB, 64)
    y = y_ref[...][:, :, :64].reshape(RB, 64)
    dinv = _deg_dinv(dp_ref)
    h = jnp.maximum(dinv * (agg + y) + b_ref[...], 0.0)
    y2 = jnp.dot(h, w_ref[...], preferred_element_type=jnp.float32) * dinv
    y2_ref[...] = _pad128(y2)


def _h2_kernel(agg_ref, y_ref, dp_ref, b_ref, h_ref):
    agg = agg_ref[...][:, :, :32].reshape(RB, 32)
    y = y_ref[...][:, :, :32].reshape(RB, 32)
    dinv = _deg_dinv(dp_ref)
    h_ref[...] = _pad128(jnp.maximum(dinv * (agg + y) + b_ref[...], 0.0))


def _mlp_kernel(gp_ref, w1, b1, w2, b2, w3, b3, w4, b4, o_ref):
    g = gp_ref[...][:, :, :32].reshape(K_POOL, 32)[:G]
    g = jnp.maximum(jnp.dot(g, w1[...], preferred_element_type=jnp.float32)
                    + b1[...], 0.0)
    g = jnp.maximum(jnp.dot(g, w2[...], preferred_element_type=jnp.float32)
                    + b2[...], 0.0)
    g = jnp.maximum(jnp.dot(g, w3[...], preferred_element_type=jnp.float32)
                    + b3[...], 0.0)
    o_ref[...] = jnp.dot(g, w4[...], preferred_element_type=jnp.float32) + b4[...]


def _full(shape):
    return pl.BlockSpec(shape, lambda i: tuple(0 for _ in shape))


def _rows(shape):
    return pl.BlockSpec(shape, lambda i: (i,) + tuple(0 for _ in shape[1:]))


def kernel(x, edge_index, batch, W1, b1, W2, b2, RW1, Rb1, RW2, Rb2, RW3, Rb3, RW4, Rb4):
    f32 = jnp.float32
    ei32 = edge_index.astype(jnp.int32)
    batch32 = batch.astype(jnp.int32)

    src_rows, dst_rows = pl.pallas_call(
        _idx_kernel, grid=(ROWS_E // 128,),
        in_specs=[pl.BlockSpec((2, 128, 128),
                               lambda i: (0, jnp.minimum(i, E // ECOLS), 0))],
        out_specs=[_rows((128, 128)), _rows((128, 128))],
        out_shape=[jax.ShapeDtypeStruct((ROWS_E, 128), jnp.int32),
                   jax.ShapeDtypeStruct((ROWS_E, 128), jnp.int32)],
    )(ei32.reshape(2, E // 128, 128))

    n_pad_p = ROWS_P * 128 - N
    iot_p = lax.iota(jnp.int32, n_pad_p)
    pdst_rows = jnp.concatenate(
        [batch32, (G + 8) + (iot_p % 16)]).reshape(ROWS_P, 128)

    deg_pass = _sc_pass("deg", ROWS_E, 1, K_ACC, 2)
    agg4_pass = _sc_pass("edge", ROWS_E, 2, K_ACC, 4)
    agg2_pass = _sc_pass("edge", ROWS_E, 1, K_ACC, 2)
    pool_pass = _sc_pass("pool", ROWS_P, 1, K_POOL, 2)

    deg_parts = deg_pass(dst_rows)
    deg_r = deg_parts.reshape(NP // 8, 8, 128)
    xw1 = pl.pallas_call(
        _mm1_kernel, grid=(NB,),
        in_specs=[_rows((RB, 47)), _full((47, 64))],
        out_specs=_rows((RB, 64)),
        out_shape=jax.ShapeDtypeStruct((NP, 64), f32),
    )(x, W1)

    y1p = pl.pallas_call(
        _prep_kernel, grid=(NB,),
        in_specs=[_rows((TB, 8, 128)), _rows((RB, 64))],
        out_specs=_rows((TB, 8, 128)),
        out_shape=jax.ShapeDtypeStruct((NP // 8, 8, 128), f32),
    )(deg_r, xw1)

    t1 = y1p.reshape(NP * 8, 16)
    agg1 = agg4_pass(t1, src_rows, dst_rows)
    y2p = pl.pallas_call(
        _layer1_kernel, grid=(NB,),
        in_specs=[_rows((TB, 8, 128)), _rows((TB, 8, 128)),
                  _rows((TB, 8, 128)), _full((1, 64)), _full((64, 32))],
        out_specs=_rows((TB, 8, 128)),
        out_shape=jax.ShapeDtypeStruct((NP // 8, 8, 128), f32),
    )(agg1.reshape(NP // 8, 8, 128), y1p, deg_r, b1.reshape(1, 64), W2)

    t2 = y2p.reshape(NP * 8, 16)
    agg2 = agg2_pass(t2, src_rows, dst_rows)
    h2p = pl.pallas_call(
        _h2_kernel, grid=(NB,),
        in_specs=[_rows((TB, 8, 128)), _rows((TB, 8, 128)),
                  _rows((TB, 8, 128)), _full((1, 32))],
        out_specs=_rows((TB, 8, 128)),
        out_shape=jax.ShapeDtypeStruct((NP // 8, 8, 128), f32),
    )(agg2.reshape(NP // 8, 8, 128), y2p, deg_r, b2.reshape(1, 32))

    tp = h2p.reshape(NP * 8, 16)
    gp = pool_pass(tp, pdst_rows)
    out = pl.pallas_call(
        _mlp_kernel, grid=(1,),
        in_specs=[_full((K_POOL // 8, 8, 128)),
                  _full((32, 32)), _full((1, 32)),
                  _full((32, 16)), _full((1, 16)),
                  _full((16, 8)), _full((1, 8)),
                  _full((8, 1)), _full((1, 1))],
        out_specs=_full((G, 1)),
        out_shape=jax.ShapeDtypeStruct((G, 1), f32),
    )(gp.reshape(K_POOL // 8, 8, 128), RW1, Rb1.reshape(1, 32), RW2,
      Rb2.reshape(1, 16), RW3, Rb3.reshape(1, 8), RW4, Rb4.reshape(1, 1))
    return out


# speedup vs baseline: 1.1134x; 1.0217x over previous
import jax
import jax.numpy as jnp
from jax import lax
from jax.experimental import pallas as pl
from jax.experimental.pallas import tpu as pltpu
from jax.experimental.pallas import tpu_sc as plsc

N = 100000
E = 1600000
G = 1000

NC = 2
NS = 16
LANES = 16

K_ACC = 100352
ZROWS = 64
KCH = 5
ROWS_E = 12800
ROWS_P = 800
K_POOL = 1024

_mesh = plsc.VectorSubcoreMesh(core_axis_name="c", subcore_axis_name="s")


def _sc_pass(mode, rows, spc, k_acc, out_s, spc_base=0):
    gather = mode != "deg"
    stride = 8
    if mode == "deg":
        rows_per_tile = rows // (NC * NS)
    else:
        rows_per_tile = rows // NS
    n_chunks = rows_per_tile // KCH
    assert rows_per_tile % KCH == 0
    stripe = k_acc // NS
    n_zcopy = stripe // ZROWS
    assert stripe % ZROWS == 0

    scratch = [
        pltpu.VMEM((3, KCH, 128), jnp.int32),
        pltpu.VMEM((min(ZROWS, stripe), LANES), jnp.float32),
        pltpu.SemaphoreType.DMA,
        pltpu.SemaphoreType.DMA,
    ]
    if gather:
        scratch += [
            pltpu.VMEM((2, KCH, 128), jnp.int32),
            pltpu.VMEM((2, KCH, 128, LANES), jnp.float32),
            pltpu.SemaphoreType.DMA,
        ]
    else:
        scratch += [pltpu.VMEM((128, LANES), jnp.float32)]
    scratch.append(pltpu.VMEM_SHARED((k_acc, LANES), jnp.float32))

    del out_s
    out_type = jax.ShapeDtypeStruct((k_acc, 8, LANES), jnp.float32)


    def body(*refs):
        if mode == "edge":
            table, srcr, dstr, out = refs[:4]
            dbuf, zbuf, isem, ssem, sbuf, rbuf, gsem, acc = refs[4:]
        elif mode == "pool":
            table, dstr, out = refs[:3]
            dbuf, zbuf, isem, ssem, sbuf, rbuf, gsem, acc = refs[3:]
        else:
            dstr, out = refs[:2]
            dbuf, zbuf, isem, ssem, obuf, acc = refs[2:]
        cid = lax.axis_index("c")
        sid = lax.axis_index("s")
        iota16 = lax.iota(jnp.int32, 16)

        zn = min(ZROWS, stripe)
        @pl.loop(0, zn)
        def _(i):
            zbuf[i, :] = jnp.zeros((LANES,), jnp.float32)
        if not gather:
            @pl.loop(0, 128)
            def _(i):
                obuf[i, :] = jnp.ones((LANES,), jnp.float32)

        if mode == "deg":
            row0 = (cid * NS + sid) * rows_per_tile
        else:
            row0 = sid * rows_per_tile

        def fire_idx(t):
            p3 = lax.rem(t, 3)
            rbase = row0 + t * KCH
            h = [pltpu.async_copy(dstr.at[pl.ds(rbase, KCH)], dbuf.at[p3], isem)]
            if mode == "edge":
                p2 = lax.rem(t, 2)
                h.append(pltpu.async_copy(srcr.at[pl.ds(rbase, KCH)],
                                          sbuf.at[p2], isem))
            return h

        def wait_idx(t):
            p3 = lax.rem(t, 3)
            rbase = row0 + t * KCH
            pltpu.make_async_copy(dstr.at[pl.ds(rbase, KCH)], dbuf.at[p3],
                                  isem).wait()
            if mode == "edge":
                p2 = lax.rem(t, 2)
                pltpu.make_async_copy(srcr.at[pl.ds(rbase, KCH)], sbuf.at[p2],
                                      isem).wait()

        def stage_b(t, s):
            wait_idx(t)
            if not gather:
                return
            p2 = lax.rem(t, 2)
            if mode == "edge":
                for j in range(KCH):
                    for g in range(8):
                        v = sbuf[p2, j, pl.ds(g * 16, 16)]
                        sbuf[p2, j, pl.ds(g * 16, 16)] = v * stride + s
            else:
                rbase = row0 + t * KCH
                for j in range(KCH):
                    for g in range(8):
                        vid = (rbase + j) * 128 + g * 16 + iota16
                        vid = jnp.minimum(vid, N - 1)
                        sbuf[p2, j, pl.ds(g * 16, 16)] = vid * stride + s
            for j in range(KCH):
                pltpu.async_copy(table.at[sbuf.at[p2, j]], rbuf.at[p2, j], gsem)

        def stage_c(t):
            p2 = lax.rem(t, 2)
            p3 = lax.rem(t, 3)
            for j in range(KCH):
                if gather:
                    pltpu.make_async_copy(table.at[sbuf.at[p2, j]],
                                          rbuf.at[p2, j], gsem).wait()
                    src_rows = rbuf.at[p2, j]
                else:
                    src_rows = obuf
                pltpu.async_copy(src_rows, acc.at[dbuf.at[p3, j]], ssem,
                                 add=True)

        def stage_d(t):
            p2 = lax.rem(t, 2)
            p3 = lax.rem(t, 3)
            for j in range(KCH):
                src_rows = rbuf.at[p2, j] if gather else obuf
                pltpu.make_async_copy(src_rows, acc.at[dbuf.at[p3, j]],
                                      ssem).wait()

        for sl in range(spc):
            s = spc_base + (cid * spc + sl) if mode == "edge" else cid

            @pl.loop(0, n_zcopy)
            def _(i):
                pltpu.sync_copy(zbuf, acc.at[pl.ds(sid * stripe + i * zn, zn)])
            plsc.subcore_barrier()

            fire_idx(0)

            @pl.loop(0, n_chunks + 2)
            def _(c):
                @pl.when(c >= 2)
                def _():
                    stage_d(c - 2)
                if gather:
                    @pl.when((c >= 1) & (c <= n_chunks))
                    def _():
                        stage_c(c - 1)
                @pl.when(c + 1 <= n_chunks - 1)
                def _():
                    fire_idx(c + 1)
                @pl.when(c <= n_chunks - 1)
                def _():
                    if gather:
                        stage_b(c, s)
                    else:
                        wait_idx(c)
                        stage_c(c)
            plsc.subcore_barrier()

            @pl.loop(0, n_zcopy)
            def _(i):
                off = sid * stripe + i * zn
                pltpu.sync_copy(acc.at[pl.ds(off, zn)],
                                out.at[pl.ds(off, zn), s])
            plsc.subcore_barrier()

    return pl.kernel(
        body, out_type=out_type, mesh=_mesh, scratch_types=scratch,
        compiler_params=pltpu.CompilerParams(use_tc_tiling_on_sc=False),
    )


NP = K_ACC
RB = 3584
NB = NP // RB
TB = RB // 8
ECOLS = 16384


def _idx_kernel(e_ref, src_ref, dst_ref):
    i = pl.program_id(0)
    f = (i * ECOLS
         + lax.broadcasted_iota(jnp.int32, (128, 128), 0) * 128
         + lax.broadcasted_iota(jnp.int32, (128, 128), 1))
    mask = f < E
    src_ref[...] = jnp.where(mask, e_ref[0].reshape(128, 128),
                             lax.rem(f, N))
    dst_ref[...] = jnp.where(mask, e_ref[1].reshape(128, 128),
                             N + (f & 15))


def _deg_dinv(dp_ref):
    m = dp_ref[...][:, :, :32].reshape(RB, 32)
    return lax.rsqrt(m[:, 0:1] + m[:, 16:17] + 1.0)


def _pad128(v):
    r, c = v.shape
    return jnp.concatenate(
        [v.reshape(r // 8, 8, c),
         jnp.zeros((r // 8, 8, 128 - c), jnp.float32)], axis=2)


def _mm1_kernel(x_ref, w_ref, o_ref):
    o_ref[...] = jnp.dot(x_ref[...], w_ref[...],
                         preferred_element_type=jnp.float32)


def _prep_kernel(dp_ref, xw_ref, y_ref):
    dinv = _deg_dinv(dp_ref)
    y_ref[...] = _pad128(xw_ref[...] * dinv)


def _layer1_kernel(agg_ref, y_ref, dp_ref, b_ref, w_ref, y2_ref):
    agg = agg_ref[...][:, :, :64].reshape(RB, 64)
    y = y_ref[...][:, :, :64].reshape(RB, 64)
    dinv = _deg_dinv(dp_ref)
    h = jnp.maximum(dinv * (agg + y) + b_ref[...], 0.0)
    y2 = jnp.dot(h, w_ref[...], preferred_element_type=jnp.float32) * dinv
    y2_ref[...] = _pad128(y2)


def _h2_kernel(agg_ref, y_ref, dp_ref, b_ref, h_ref):
    agg = agg_ref[...][:, :, :32].reshape(RB, 32)
    y = y_ref[...][:, :, :32].reshape(RB, 32)
    dinv = _deg_dinv(dp_ref)
    h_ref[...] = _pad128(jnp.maximum(dinv * (agg + y) + b_ref[...], 0.0))


def _mlp_kernel(gp_ref, w1, b1, w2, b2, w3, b3, w4, b4, o_ref):
    g = gp_ref[...][:, :, :32].reshape(K_POOL, 32)[:G]
    g = jnp.maximum(jnp.dot(g, w1[...], preferred_element_type=jnp.float32)
                    + b1[...], 0.0)
    g = jnp.maximum(jnp.dot(g, w2[...], preferred_element_type=jnp.float32)
                    + b2[...], 0.0)
    g = jnp.maximum(jnp.dot(g, w3[...], preferred_element_type=jnp.float32)
                    + b3[...], 0.0)
    o_ref[...] = jnp.dot(g, w4[...], preferred_element_type=jnp.float32) + b4[...]


def _full(shape):
    return pl.BlockSpec(shape, lambda i: tuple(0 for _ in shape))


def _rows(shape):
    return pl.BlockSpec(shape, lambda i: (i,) + tuple(0 for _ in shape[1:]))


def kernel(x, edge_index, batch, W1, b1, W2, b2, RW1, Rb1, RW2, Rb2, RW3, Rb3, RW4, Rb4):
    f32 = jnp.float32
    ei32 = edge_index.astype(jnp.int32)
    batch32 = batch.astype(jnp.int32)

    src_rows, dst_rows = pl.pallas_call(
        _idx_kernel, grid=(ROWS_E // 128,),
        in_specs=[pl.BlockSpec((2, 128, 128),
                               lambda i: (0, jnp.minimum(i, E // ECOLS), 0))],
        out_specs=[_rows((128, 128)), _rows((128, 128))],
        out_shape=[jax.ShapeDtypeStruct((ROWS_E, 128), jnp.int32),
                   jax.ShapeDtypeStruct((ROWS_E, 128), jnp.int32)],
    )(ei32.reshape(2, E // 128, 128))

    n_pad_p = ROWS_P * 128 - N
    iot_p = lax.iota(jnp.int32, n_pad_p)
    pdst_rows = jnp.concatenate(
        [batch32, (G + 8) + (iot_p % 16)]).reshape(ROWS_P, 128)

    deg_pass = _sc_pass("deg", ROWS_E, 1, K_ACC, 2)
    agg4_pass = _sc_pass("edge", ROWS_E, 2, K_ACC, 4)
    agg2_pass = _sc_pass("edge", ROWS_E, 1, K_ACC, 2)
    pool_pass = _sc_pass("pool", ROWS_P, 1, K_POOL, 2)

    deg_parts = deg_pass(dst_rows)
    deg_r = deg_parts.reshape(NP // 8, 8, 128)
    xw1 = pl.pallas_call(
        _mm1_kernel, grid=(NB,),
        in_specs=[_rows((RB, 47)), _full((47, 64))],
        out_specs=_rows((RB, 64)),
        out_shape=jax.ShapeDtypeStruct((NP, 64), f32),
    )(x, W1)

    y1p = pl.pallas_call(
        _prep_kernel, grid=(NB,),
        in_specs=[_rows((TB, 8, 128)), _rows((RB, 64))],
        out_specs=_rows((TB, 8, 128)),
        out_shape=jax.ShapeDtypeStruct((NP // 8, 8, 128), f32),
    )(deg_r, xw1)

    t1 = y1p.reshape(NP * 8, 16)
    agg1 = agg4_pass(t1, src_rows, dst_rows)
    y2p = pl.pallas_call(
        _layer1_kernel, grid=(NB,),
        in_specs=[_rows((TB, 8, 128)), _rows((TB, 8, 128)),
                  _rows((TB, 8, 128)), _full((1, 64)), _full((64, 32))],
        out_specs=_rows((TB, 8, 128)),
        out_shape=jax.ShapeDtypeStruct((NP // 8, 8, 128), f32),
    )(agg1.reshape(NP // 8, 8, 128), y1p, deg_r, b1.reshape(1, 64), W2)

    t2 = y2p.reshape(NP * 8, 16)
    agg2 = agg2_pass(t2, src_rows, dst_rows)
    h2p = pl.pallas_call(
        _h2_kernel, grid=(NB,),
        in_specs=[_rows((TB, 8, 128)), _rows((TB, 8, 128)),
                  _rows((TB, 8, 128)), _full((1, 32))],
        out_specs=_rows((TB, 8, 128)),
        out_shape=jax.ShapeDtypeStruct((NP // 8, 8, 128), f32),
    )(agg2.reshape(NP // 8, 8, 128), y2p, deg_r, b2.reshape(1, 32))

    tp = h2p.reshape(NP * 8, 16)
    gp = pool_pass(tp, pdst_rows)
    out = pl.pallas_call(
        _mlp_kernel, grid=(1,),
        in_specs=[_full((K_POOL // 8, 8, 128)),
                  _full((32, 32)), _full((1, 32)),
                  _full((32, 16)), _full((1, 16)),
                  _full((16, 8)), _full((1, 8)),
                  _full((8, 1)), _full((1, 1))],
        out_specs=_full((G, 1)),
        out_shape=jax.ShapeDtypeStruct((G, 1), f32),
    )(gp.reshape(K_POOL // 8, 8, 128), RW1, Rb1.reshape(1, 32), RW2,
      Rb2.reshape(1, 16), RW3, Rb3.reshape(1, 8), RW4, Rb4.reshape(1, 1))
    return out
